# baseline (speedup 1.0000x reference)
import functools

import jax
import jax.numpy as jnp
from jax import lax
from jax.experimental import pallas as pl
from jax.experimental.pallas import tpu as pltpu
from jax.experimental.pallas import tpu_sc as plsc

EMBED_DIM = 64
SEQ = 200
NVIS = 4096
TU = 2
RU = TU * 128
NU = SEQ // TU
JT, JS = EMBED_DIM // 8, 8
VT, VL = NVIS // 128, 128
TRN = JT * JS * VL


def _emb_lookup(idx4, table):
    info = plsc.get_sparse_core_info()
    NC = info.num_cores

    mesh = plsc.VectorSubcoreMesh(core_axis_name="c", subcore_axis_name="s")

    @functools.partial(
        pl.kernel,
        mesh=mesh,
        out_type=jax.ShapeDtypeStruct((SEQ, JT, VT, JS, VL), jnp.float32),
        scratch_types=[
            pltpu.VMEM((RU,), jnp.int32),
            pltpu.VMEM((RU,), jnp.int32),
            pltpu.VMEM((RU, EMBED_DIM), jnp.float32),
            pltpu.VMEM((RU, EMBED_DIM), jnp.float32),
            pltpu.VMEM((TU * JT * JS, VL + 1), jnp.float32),
            pltpu.VMEM((TU * JT * JS, VL + 1), jnp.float32),
            pltpu.SemaphoreType.DMA,
            pltpu.SemaphoreType.DMA,
            pltpu.SemaphoreType.DMA,
            pltpu.SemaphoreType.DMA,
            pltpu.SemaphoreType.DMA,
            pltpu.SemaphoreType.DMA,
        ],
        compiler_params=pltpu.CompilerParams(use_tc_tiling_on_sc=False,
                                             needs_layout_passes=False),
    )
    def emb_kernel(idx_hbm, table_hbm, out_hbm,
                   idx0, idx1, rows0, rows1, tr0, tr1,
                   si0, si1, sg0, sg1, ss0, ss1):
        idx_v = (idx0, idx1)
        rows_v = (rows0, rows1)
        tr_v = (tr0, tr1)
        si = (si0, si1)
        sg = (sg0, sg1)
        ss = (ss0, ss1)
        wid = lax.axis_index("s") * NC + lax.axis_index("c")
        iota16 = lax.iota(jnp.int32, 16)
        rvec = [[iota16 + (tu * EMBED_DIM + s * 16) for s in range(4)]
                for tu in range(TU)]

        def start_idx(i, b):
            tt = i // 4
            ts = (TU * i) % 8
            for tu in range(TU):
                pltpu.async_copy(idx_hbm.at[tt, wid, ts + tu, :],
                                 idx_v[b].at[pl.ds(tu * VL, VL)], si[b])

        def wait_idx(b):
            for tu in range(TU):
                pltpu.make_async_copy(idx_hbm.at[0, 0, 0, :],
                                      idx_v[b].at[pl.ds(0, VL)], si[b]).wait()

        def start_gather(b):
            pltpu.async_copy(table_hbm.at[idx_v[b]], rows_v[b], sg[b])

        def wait_gather(b):
            pltpu.make_async_copy(table_hbm.at[idx_v[b]],
                                  rows_v[b], sg[b]).wait()

        def start_store(i, b):
            for tu in range(TU):
                for jt in range(JT):
                    pltpu.async_copy(
                        tr_v[b].at[pl.ds((tu * JT + jt) * JS, JS),
                                   pl.ds(0, VL)],
                        out_hbm.at[TU * i + tu, jt, wid], ss[b])

        def wait_store(b):
            for _ in range(TU * JT):
                pltpu.make_async_copy(
                    tr_v[b].at[pl.ds(0, JS), pl.ds(0, VL)],
                    out_hbm.at[0, 0, 0], ss[b]).wait()

        def compute(b):
            rows = rows_v[b]
            tr = tr_v[b]
            for tu in range(TU):
                @plsc.parallel_loop(0, VL, unroll=4)
                def r_body(r):
                    cvec = jnp.full((16,), r, jnp.int32)
                    for s in range(4):
                        val = rows[tu * VL + r, pl.ds(s * 16, 16)]
                        plsc.store_scatter(tr, [rvec[tu][s], cvec], val)

        def step(i, b):
            o = 1 - b
            wait_idx(b)
            start_gather(b)
            wait_gather(o)
            wait_store(o)
            compute(o)
            start_idx(i + 1, o)
            start_store(i - 1, o)

        start_idx(0, 0)
        start_idx(1, 1)
        wait_idx(0)
        start_gather(0)
        wait_idx(1)
        start_gather(1)
        wait_gather(0)
        compute(0)
        start_idx(2, 0)
        start_store(0, 0)
        wait_idx(0)
        start_gather(0)
        wait_gather(1)
        compute(1)
        start_idx(3, 1)
        start_store(1, 1)
        def pair(g, carry):
            i = 3 + 2 * g
            step(i, 1)
            step(i + 1, 0)
            return carry
        lax.fori_loop(0, (NU - 4) // 2, pair, 0)
        wait_idx(1)
        start_gather(1)
        wait_gather(0)
        wait_store(0)
        compute(0)
        start_store(NU - 2, 0)
        wait_gather(1)
        wait_store(1)
        compute(1)
        start_store(NU - 1, 1)
        wait_store(0)
        wait_store(1)

    return emb_kernel(idx4, table)


def kernel(visit_node_type, table):
    NV, S = visit_node_type.shape
    idx4 = (visit_node_type.astype(jnp.int32).T
            .reshape(S // 8, 8, NV // VL, VL).transpose(0, 2, 1, 3))
    out5 = _emb_lookup(idx4, table)
    return out5.transpose(2, 4, 0, 1, 3).reshape(NV, S, EMBED_DIM)


# speedup vs baseline: 1.0174x; 1.0174x over previous
import functools

import jax
import jax.numpy as jnp
from jax import lax
from jax.experimental import pallas as pl
from jax.experimental.pallas import tpu as pltpu
from jax.experimental.pallas import tpu_sc as plsc

EMBED_DIM = 64
SEQ = 200
NVIS = 4096
TU = 2
RU = TU * 128
NU = SEQ // TU
JT, JS = EMBED_DIM // 8, 8
VT, VL = NVIS // 128, 128
TRN = JT * JS * VL


def _emb_lookup(idx4, table):
    info = plsc.get_sparse_core_info()
    NC = info.num_cores

    mesh = plsc.VectorSubcoreMesh(core_axis_name="c", subcore_axis_name="s")

    @functools.partial(
        pl.kernel,
        mesh=mesh,
        out_type=jax.ShapeDtypeStruct((SEQ, JT, VT, JS, VL), jnp.float32),
        scratch_types=[
            pltpu.VMEM((RU,), jnp.int32),
            pltpu.VMEM((RU,), jnp.int32),
            pltpu.VMEM((RU, EMBED_DIM), jnp.float32),
            pltpu.VMEM((RU, EMBED_DIM), jnp.float32),
            pltpu.VMEM((TU * JT, JS, VL + 1), jnp.float32),
            pltpu.VMEM((TU * JT, JS, VL + 1), jnp.float32),
            pltpu.SemaphoreType.DMA,
            pltpu.SemaphoreType.DMA,
            pltpu.SemaphoreType.DMA,
            pltpu.SemaphoreType.DMA,
            pltpu.SemaphoreType.DMA,
            pltpu.SemaphoreType.DMA,
        ],
        compiler_params=pltpu.CompilerParams(use_tc_tiling_on_sc=False,
                                             needs_layout_passes=False),
    )
    def emb_kernel(idx_hbm, table_hbm, out_hbm,
                   idx0, idx1, rows0, rows1, tr0, tr1,
                   si0, si1, sg0, sg1, ss0, ss1):
        idx_v = (idx0, idx1)
        rows_v = (rows0, rows1)
        tr_v = (tr0, tr1)
        si = (si0, si1)
        sg = (sg0, sg1)
        ss = (ss0, ss1)
        wid = lax.axis_index("s") * NC + lax.axis_index("c")
        iota16 = lax.iota(jnp.int32, 16)
        jtvec = [[(iota16 + (tu * EMBED_DIM + s * 16)) // JS for s in range(4)]
                 for tu in range(TU)]
        jsvec = [iota16 % JS + 0 * s for s in range(1)]

        def start_idx(i, b):
            tt = i // 4
            ts = (TU * i) % 8
            for tu in range(TU):
                pltpu.async_copy(idx_hbm.at[tt, wid, ts + tu, :],
                                 idx_v[b].at[pl.ds(tu * VL, VL)], si[b])

        def wait_idx(b):
            for tu in range(TU):
                pltpu.make_async_copy(idx_hbm.at[0, 0, 0, :],
                                      idx_v[b].at[pl.ds(0, VL)], si[b]).wait()

        def start_gather(b):
            pltpu.async_copy(table_hbm.at[idx_v[b]], rows_v[b], sg[b])

        def wait_gather(b):
            pltpu.make_async_copy(table_hbm.at[idx_v[b]],
                                  rows_v[b], sg[b]).wait()

        def start_store(i, b):
            for tu in range(TU):
                pltpu.async_copy(
                    tr_v[b].at[pl.ds(tu * JT, JT), :, pl.ds(0, VL)],
                    out_hbm.at[TU * i + tu, :, wid], ss[b])

        def wait_store(b):
            for _ in range(TU):
                pltpu.make_async_copy(
                    tr_v[b].at[pl.ds(0, JT), :, pl.ds(0, VL)],
                    out_hbm.at[0, :, 0], ss[b]).wait()

        def compute(b):
            rows = rows_v[b]
            tr = tr_v[b]
            for tu in range(TU):
                @plsc.parallel_loop(0, VL, unroll=4)
                def r_body(r):
                    cvec = jnp.full((16,), r, jnp.int32)
                    for s in range(4):
                        val = rows[tu * VL + r, pl.ds(s * 16, 16)]
                        plsc.store_scatter(
                            tr, [jtvec[tu][s], jsvec[0], cvec], val)

        def step(i, b):
            o = 1 - b
            wait_idx(b)
            start_gather(b)
            wait_gather(o)
            wait_store(o)
            compute(o)
            start_idx(i + 1, o)
            start_store(i - 1, o)

        start_idx(0, 0)
        start_idx(1, 1)
        wait_idx(0)
        start_gather(0)
        wait_idx(1)
        start_gather(1)
        wait_gather(0)
        compute(0)
        start_idx(2, 0)
        start_store(0, 0)
        wait_idx(0)
        start_gather(0)
        wait_gather(1)
        compute(1)
        start_idx(3, 1)
        start_store(1, 1)
        def pair(g, carry):
            i = 3 + 2 * g
            step(i, 1)
            step(i + 1, 0)
            return carry
        lax.fori_loop(0, (NU - 4) // 2, pair, 0)
        wait_idx(1)
        start_gather(1)
        wait_gather(0)
        wait_store(0)
        compute(0)
        start_store(NU - 2, 0)
        wait_gather(1)
        wait_store(1)
        compute(1)
        start_store(NU - 1, 1)
        wait_store(0)
        wait_store(1)

    return emb_kernel(idx4, table)


def kernel(visit_node_type, table):
    NV, S = visit_node_type.shape
    idx4 = (visit_node_type.astype(jnp.int32).T
            .reshape(S // 8, 8, NV // VL, VL).transpose(0, 2, 1, 3))
    out5 = _emb_lookup(idx4, table)
    return out5.transpose(2, 4, 0, 1, 3).reshape(NV, S, EMBED_DIM)
